# baseline (speedup 1.0000x reference)
import jax
import jax.numpy as jnp
from jax import lax
from jax.experimental import pallas as pl
from jax.experimental.pallas import tpu as pltpu
from jax.experimental.pallas import tpu_sc as plsc

NC = 2
NS = 16
NW = NC * NS
CH = 128
DE = 16


def _dot_t(a, w):
    return lax.dot_general(a, w, (((1,), (1,)), ((), ())),
                           preferred_element_type=jnp.float32)


def _project_first(x, wn, ws, grid):
    NR, D = x.shape
    DO = wn.shape[0]
    BR = NR // grid

    def body(x_ref, wn_ref, ws_ref, q_ref, s_ref):
        h = x_ref[...]
        q = _dot_t(h, wn_ref[...])
        q_ref[...] = jnp.concatenate(
            [q, jnp.ones((BR, DE), jnp.float32)], axis=1)
        s_ref[...] = _dot_t(h, ws_ref[...])

    return pl.pallas_call(
        body,
        grid=(grid,),
        in_specs=[
            pl.BlockSpec((BR, D), lambda i: (i, 0)),
            pl.BlockSpec((DO, D), lambda i: (0, 0)),
            pl.BlockSpec((DO, D), lambda i: (0, 0)),
        ],
        out_specs=[
            pl.BlockSpec((BR, DO + DE), lambda i: (i, 0)),
            pl.BlockSpec((BR, DO), lambda i: (i, 0)),
        ],
        out_shape=[
            jax.ShapeDtypeStruct((NR, DO + DE), jnp.float32),
            jax.ShapeDtypeStruct((NR, DO), jnp.float32),
        ],
    )(x, wn, ws)


def _combine_project(s1, aggp, b, wn, ws, grid):
    NR, DH = s1.shape
    DO = wn.shape[0]
    BR = NR // grid

    def body(s1_ref, agg_ref, b_ref, wn_ref, ws_ref, q_ref, s_ref, r_ref):
        agg = agg_ref[0, :, 0:DH] + agg_ref[1, :, 0:DH]
        deg = agg_ref[0, :, DH:DH + 1] + agg_ref[1, :, DH:DH + 1]
        recip = 1.0 / jnp.maximum(deg, 1.0)
        h = s1_ref[...] + agg * recip + b_ref[...]
        h = jnp.maximum(h, 0.0)
        q_ref[...] = _dot_t(h, wn_ref[...])
        s_ref[...] = _dot_t(h, ws_ref[...])
        r_ref[...] = jnp.broadcast_to(recip, (BR, 8))

    return pl.pallas_call(
        body,
        grid=(grid,),
        in_specs=[
            pl.BlockSpec((BR, DH), lambda i: (i, 0)),
            pl.BlockSpec((NC, BR, DH + DE), lambda i: (0, i, 0)),
            pl.BlockSpec((1, DH), lambda i: (0, 0)),
            pl.BlockSpec((DO, DH), lambda i: (0, 0)),
            pl.BlockSpec((DO, DH), lambda i: (0, 0)),
        ],
        out_specs=[
            pl.BlockSpec((BR, DO), lambda i: (i, 0)),
            pl.BlockSpec((BR, DO), lambda i: (i, 0)),
            pl.BlockSpec((BR, 8), lambda i: (i, 0)),
        ],
        out_shape=[
            jax.ShapeDtypeStruct((NR, DO), jnp.float32),
            jax.ShapeDtypeStruct((NR, DO), jnp.float32),
            jax.ShapeDtypeStruct((NR, 8), jnp.float32),
        ],
    )(s1, aggp, b, wn, ws)


def _combine_final(s2, aggp, recip, b, grid):
    NR, DO = s2.shape
    BR = NR // grid

    def body(s2_ref, agg_ref, r_ref, b_ref, o_ref):
        agg = agg_ref[0] + agg_ref[1]
        o_ref[...] = s2_ref[...] + agg * r_ref[:, 0:1] + b_ref[...]

    return pl.pallas_call(
        body,
        grid=(grid,),
        in_specs=[
            pl.BlockSpec((BR, DO), lambda i: (i, 0)),
            pl.BlockSpec((NC, BR, DO), lambda i: (0, i, 0)),
            pl.BlockSpec((BR, 8), lambda i: (i, 0)),
            pl.BlockSpec((1, DO), lambda i: (0, 0)),
        ],
        out_specs=pl.BlockSpec((BR, DO), lambda i: (i, 0)),
        out_shape=jax.ShapeDtypeStruct((NR, DO), jnp.float32),
    )(s2, aggp, recip, b)


def _sc_segment_sum(q, src3, dst3, zrow, NP):
    D = q.shape[1]
    C = src3.shape[1]
    RPS = NP // NS
    mesh = plsc.VectorSubcoreMesh(core_axis_name="c", subcore_axis_name="s")

    def body(q_hbm, src_hbm, dst_hbm, z_hbm, agg_out,
             src_v, dst_v, buf, agg_sh, sem_g, sem_s):
        c = lax.axis_index("c")
        s = lax.axis_index("s")
        wid = s * NC + c
        r0 = s * RPS
        pltpu.sync_copy(z_hbm, agg_sh.at[pl.ds(r0, RPS)])
        pltpu.sync_copy(src_hbm.at[wid], src_v)
        pltpu.sync_copy(dst_hbm.at[wid], dst_v)
        plsc.subcore_barrier()

        pltpu.async_copy(q_hbm.at[src_v.at[0]], buf.at[0], sem_g)

        def step(j, carry):
            slot = lax.rem(j, 2)
            nslot = lax.rem(j + 1, 2)
            pltpu.make_async_copy(
                q_hbm.at[src_v.at[j]], buf.at[slot], sem_g).wait()

            @pl.when(j >= 1)
            def _():
                pltpu.make_async_copy(
                    buf.at[nslot], agg_sh.at[dst_v.at[j - 1]], sem_s).wait()

            @pl.when(j + 1 < C)
            def _():
                pltpu.async_copy(
                    q_hbm.at[src_v.at[j + 1]], buf.at[nslot], sem_g)

            pltpu.async_copy(
                buf.at[slot], agg_sh.at[dst_v.at[j]], sem_s, add=True)
            return carry

        lax.fori_loop(0, C, step, 0)
        pltpu.make_async_copy(
            buf.at[lax.rem(C - 1, 2)],
            agg_sh.at[dst_v.at[C - 1]], sem_s).wait()

        plsc.subcore_barrier()
        pltpu.sync_copy(agg_sh.at[pl.ds(r0, RPS)],
                        agg_out.at[c, pl.ds(r0, RPS)])

    f = pl.kernel(
        body,
        out_type=jax.ShapeDtypeStruct((NC, NP, D), jnp.float32),
        mesh=mesh,
        scratch_types=(
            pltpu.VMEM((C, CH), jnp.int32),
            pltpu.VMEM((C, CH), jnp.int32),
            pltpu.VMEM((2, CH, D), jnp.float32),
            pltpu.VMEM_SHARED((NP, D), jnp.float32),
            pltpu.SemaphoreType.DMA,
            pltpu.SemaphoreType.DMA,
        ),
        compiler_params=pltpu.CompilerParams(use_tc_tiling_on_sc=False),
    )
    return f(q, src3, dst3, zrow)


def kernel(x, edge_index, W_self1, W_neigh1, b1, W_self2, W_neigh2, b2):
    N, D_IN = x.shape
    E = edge_index.shape[1]
    D_H = W_self1.shape[0]
    D_OUT = W_self2.shape[0]

    NP = ((N + 1024) // 1024) * 1024
    RPS = NP // NS
    C = -(-E // (NW * CH))
    EPAD = NW * C * CH

    grid = next(g for g in (10, 8, 5, 4, 2, 1)
                if N % g == 0 and (N // g) % 8 == 0)

    pad = EPAD - E
    src3 = jnp.concatenate(
        [edge_index[0], jnp.zeros((pad,), jnp.int32)]).reshape(NW, C, CH)
    dst3 = jnp.concatenate(
        [edge_index[1], jnp.full((pad,), N, jnp.int32)]).reshape(NW, C, CH)
    zrow1 = jnp.zeros((RPS, D_H + DE), jnp.float32)
    zrow2 = jnp.zeros((RPS, D_OUT), jnp.float32)
    b1r = b1.reshape(1, D_H)
    b2r = b2.reshape(1, D_OUT)

    q1e, s1 = _project_first(x, W_neigh1, W_self1, grid)
    aggp1 = _sc_segment_sum(q1e, src3, dst3, zrow1, NP)
    q2, s2, recip = _combine_project(s1, aggp1, b1r, W_neigh2, W_self2, grid)

    aggp2 = _sc_segment_sum(q2, src3, dst3, zrow2, NP)
    return _combine_final(s2, aggp2, recip, b2r, grid)


# speedup vs baseline: 1.2221x; 1.2221x over previous
import jax
import jax.numpy as jnp
from jax import lax
from jax.experimental import pallas as pl
from jax.experimental.pallas import tpu as pltpu
from jax.experimental.pallas import tpu_sc as plsc

NC = 2
NS = 16
CH = 128
DE = 16
SLOW_FRAC = 0.363


def _dot_t(a, w):
    return lax.dot_general(a, w, (((1,), (1,)), ((), ())),
                           preferred_element_type=jnp.float32)


def _project_first(x, wn, ws, grid=8):
    NP, D = x.shape
    DO = wn.shape[0]
    BR = NP // grid

    def body(x_ref, wn_ref, ws_ref, q_ref, s_ref):
        h = x_ref[...]
        q = _dot_t(h, wn_ref[...])
        q_ref[...] = jnp.concatenate(
            [q, jnp.ones((BR, DE), jnp.float32)], axis=1)
        s_ref[...] = _dot_t(h, ws_ref[...])

    return pl.pallas_call(
        body,
        grid=(grid,),
        in_specs=[
            pl.BlockSpec((BR, D), lambda i: (i, 0)),
            pl.BlockSpec((DO, D), lambda i: (0, 0)),
            pl.BlockSpec((DO, D), lambda i: (0, 0)),
        ],
        out_specs=[
            pl.BlockSpec((BR, DO + DE), lambda i: (i, 0)),
            pl.BlockSpec((BR, DO), lambda i: (i, 0)),
        ],
        out_shape=[
            jax.ShapeDtypeStruct((NP, DO + DE), jnp.float32),
            jax.ShapeDtypeStruct((NP, DO), jnp.float32),
        ],
    )(x, wn, ws)


def _combine_project(s1, aggp, b, wn, ws, grid=8):
    NP, DH = s1.shape
    DO = wn.shape[0]
    BR = NP // grid

    def body(s1_ref, agg_ref, b_ref, wn_ref, ws_ref, q_ref, s_ref, r_ref):
        agg = agg_ref[0, :, 0:DH] + agg_ref[1, :, 0:DH]
        deg = agg_ref[0, :, DH:DH + 1] + agg_ref[1, :, DH:DH + 1]
        recip = 1.0 / jnp.maximum(deg, 1.0)
        h = s1_ref[...] + agg * recip + b_ref[...]
        h = jnp.maximum(h, 0.0)
        q_ref[...] = _dot_t(h, wn_ref[...])
        s_ref[...] = _dot_t(h, ws_ref[...])
        r_ref[...] = jnp.broadcast_to(recip, (BR, 8))

    return pl.pallas_call(
        body,
        grid=(grid,),
        in_specs=[
            pl.BlockSpec((BR, DH), lambda i: (i, 0)),
            pl.BlockSpec((NC, BR, DH + DE), lambda i: (0, i, 0)),
            pl.BlockSpec((1, DH), lambda i: (0, 0)),
            pl.BlockSpec((DO, DH), lambda i: (0, 0)),
            pl.BlockSpec((DO, DH), lambda i: (0, 0)),
        ],
        out_specs=[
            pl.BlockSpec((BR, DO), lambda i: (i, 0)),
            pl.BlockSpec((BR, DO), lambda i: (i, 0)),
            pl.BlockSpec((BR, 8), lambda i: (i, 0)),
        ],
        out_shape=[
            jax.ShapeDtypeStruct((NP, DO), jnp.float32),
            jax.ShapeDtypeStruct((NP, DO), jnp.float32),
            jax.ShapeDtypeStruct((NP, 8), jnp.float32),
        ],
    )(s1, aggp, b, wn, ws)


def _combine_final(s2, aggp, recip, b, grid=8):
    NP, DO = s2.shape
    BR = NP // grid

    def body(s2_ref, agg_ref, r_ref, b_ref, o_ref):
        agg = agg_ref[0] + agg_ref[1]
        o_ref[...] = s2_ref[...] + agg * r_ref[:, 0:1] + b_ref[...]

    return pl.pallas_call(
        body,
        grid=(grid,),
        in_specs=[
            pl.BlockSpec((BR, DO), lambda i: (i, 0)),
            pl.BlockSpec((NC, BR, DO), lambda i: (0, i, 0)),
            pl.BlockSpec((BR, 8), lambda i: (i, 0)),
            pl.BlockSpec((1, DO), lambda i: (0, 0)),
        ],
        out_specs=pl.BlockSpec((BR, DO), lambda i: (i, 0)),
        out_shape=jax.ShapeDtypeStruct((NP, DO), jnp.float32),
    )(s2, aggp, recip, b)


def _sc_segment_sum(q, src2, dst2, zrow, K0, K1):
    NP, D = q.shape
    KM = max(K0, K1)
    RPS = NP // NS
    mesh = plsc.VectorSubcoreMesh(core_axis_name="c", subcore_axis_name="s")

    def body(q_hbm, src_hbm, dst_hbm, z_hbm, agg_out,
             src_v, dst_v, buf, agg_sh, sem_g, sem_s):
        c = lax.axis_index("c")
        s = lax.axis_index("s")
        r0 = s * RPS
        pltpu.sync_copy(z_hbm, agg_sh.at[pl.ds(r0, RPS)])

        def run(K, base):
            pltpu.sync_copy(src_hbm.at[pl.ds(base, K)],
                            src_v.at[pl.ds(0, K)])
            pltpu.sync_copy(dst_hbm.at[pl.ds(base, K)],
                            dst_v.at[pl.ds(0, K)])
            plsc.subcore_barrier()
            pltpu.async_copy(q_hbm.at[src_v.at[0]], buf.at[0], sem_g)

            def step(j, carry):
                slot = lax.rem(j, 2)
                nslot = lax.rem(j + 1, 2)
                pltpu.make_async_copy(
                    q_hbm.at[src_v.at[j]], buf.at[slot], sem_g).wait()

                @pl.when(j >= 1)
                def _():
                    pltpu.make_async_copy(
                        buf.at[nslot],
                        agg_sh.at[dst_v.at[j - 1]], sem_s).wait()

                @pl.when(j + 1 < K)
                def _():
                    pltpu.async_copy(
                        q_hbm.at[src_v.at[j + 1]], buf.at[nslot], sem_g)

                pltpu.async_copy(
                    buf.at[slot], agg_sh.at[dst_v.at[j]], sem_s, add=True)
                return carry

            lax.fori_loop(0, K, step, 0)
            pltpu.make_async_copy(
                buf.at[lax.rem(K - 1, 2)],
                agg_sh.at[dst_v.at[K - 1]], sem_s).wait()
            plsc.subcore_barrier()
            pltpu.sync_copy(agg_sh.at[pl.ds(r0, RPS)],
                            agg_out.at[c, pl.ds(r0, RPS)])

        @pl.when(c == 0)
        def _():
            run(K0, s * K0)

        @pl.when(c == 1)
        def _():
            run(K1, NS * K0 + s * K1)

    f = pl.kernel(
        body,
        out_type=jax.ShapeDtypeStruct((NC, NP, D), jnp.float32),
        mesh=mesh,
        scratch_types=(
            pltpu.VMEM((KM, CH), jnp.int32),
            pltpu.VMEM((KM, CH), jnp.int32),
            pltpu.VMEM((2, CH, D), jnp.float32),
            pltpu.VMEM_SHARED((NP, D), jnp.float32),
            pltpu.SemaphoreType.DMA,
            pltpu.SemaphoreType.DMA,
        ),
        compiler_params=pltpu.CompilerParams(use_tc_tiling_on_sc=False),
    )
    return f(q, src2, dst2, zrow)


def kernel(x, edge_index, W_self1, W_neigh1, b1, W_self2, W_neigh2, b2):
    N, D_IN = x.shape
    E = edge_index.shape[1]
    D_H = W_self1.shape[0]
    D_OUT = W_self2.shape[0]

    NP = ((N + 1024) // 1024) * 1024
    RPS = NP // NS

    TOT = -(-E // CH)
    PAIR = -(-TOT // NS)
    K0 = max(1, round(PAIR * SLOW_FRAC))
    K1 = PAIR - K0
    EPAD = NS * (K0 + K1) * CH

    x_p = jnp.pad(x, ((0, NP - N), (0, 0)))
    pad = EPAD - E
    src2 = jnp.concatenate(
        [edge_index[0], jnp.zeros((pad,), jnp.int32)]).reshape(-1, CH)
    dst2 = jnp.concatenate(
        [edge_index[1], jnp.full((pad,), N, jnp.int32)]).reshape(-1, CH)
    zrow1 = jnp.zeros((RPS, D_H + DE), jnp.float32)
    zrow2 = jnp.zeros((RPS, D_OUT), jnp.float32)
    b1r = b1.reshape(1, D_H)
    b2r = b2.reshape(1, D_OUT)

    q1e, s1 = _project_first(x_p, W_neigh1, W_self1)
    aggp1 = _sc_segment_sum(q1e, src2, dst2, zrow1, K0, K1)
    q2, s2, recip = _combine_project(s1, aggp1, b1r, W_neigh2, W_self2)

    aggp2 = _sc_segment_sum(q2, src2, dst2, zrow2, K0, K1)
    out = _combine_final(s2, aggp2, recip, b2r)
    return out[:N]
